# baseline (speedup 1.0000x reference)
import functools

import jax
import jax.numpy as jnp
from jax import lax
from jax.experimental import pallas as pl
from jax.experimental.pallas import tpu as pltpu
from jax.experimental.pallas import tpu_sc as plsc


def _fold_params(W1, b1, W2, b2, W3, b3):
    def body(w1_ref, b1_ref, w2_ref, b2_ref, w3_ref, b3_ref, out_ref):
        w3 = w3_ref[...][:, 0]
        w23 = jnp.sum(w2_ref[...] * w3[None, :], axis=1)
        w = jnp.sum(w1_ref[...] * w23[None, :], axis=1)
        c = (jnp.sum(b1_ref[...] * w23) + jnp.sum(b2_ref[...] * w3)
             + b3_ref[0])
        out_ref[...] = jnp.concatenate(
            [w, jnp.full((16,), c, jnp.float32)])

    return pl.pallas_call(
        body,
        out_shape=jax.ShapeDtypeStruct((48,), jnp.float32),
    )(W1, b1, W2, b2, W3, b3)


def _table_dot(table, params, w_off, add_c, block_groups):
    n = table.shape[0]
    g = n // 16
    grid = (g + block_groups - 1) // block_groups
    t2 = table.reshape(g, 256)

    def body(t_ref, p_ref, out_ref):
        w = p_ref[pl.ds(w_off, 16)]
        jj = lax.broadcasted_iota(jnp.int32, (256, 16), 0)
        mm = lax.broadcasted_iota(jnp.int32, (256, 16), 1)
        wtile = jnp.tile(w, 16)
        M = jnp.where(jj >> 4 == mm, wtile[:, None], 0.0)
        s = jnp.dot(t_ref[...], M, preferred_element_type=jnp.float32)
        if add_c:
            s = s + p_ref[32]
        out_ref[...] = s

    return pl.pallas_call(
        body,
        grid=(grid,),
        in_specs=[
            pl.BlockSpec((block_groups, 256), lambda i: (i, 0)),
            pl.BlockSpec((48,), lambda i: (0,)),
        ],
        out_specs=pl.BlockSpec((block_groups, 16), lambda i: (i, 0)),
        out_shape=jax.ShapeDtypeStruct((g, 16), jnp.float32),
    )(t2, params)


def _sc_lookup(user, recipe, p2, q2):
    info = plsc.get_sparse_core_info()
    NC, NS, L = info.num_cores, info.num_subcores, info.num_lanes
    NW = NC * NS
    B = user.shape[0]
    b_per_w = B // NW
    nblk = b_per_w // L
    mesh = plsc.VectorSubcoreMesh(core_axis_name="c", subcore_axis_name="s")

    @functools.partial(
        pl.kernel,
        mesh=mesh,
        out_type=jax.ShapeDtypeStruct((B,), jnp.float32),
        compiler_params=pltpu.CompilerParams(
            needs_layout_passes=False, use_tc_tiling_on_sc=False),
        scratch_types=[
            pltpu.VMEM((b_per_w,), jnp.int32),
            pltpu.VMEM((b_per_w,), jnp.int32),
            pltpu.VMEM((b_per_w,), jnp.int32),
            pltpu.VMEM((b_per_w,), jnp.int32),
            pltpu.VMEM((b_per_w, L), jnp.float32),
            pltpu.VMEM((b_per_w, L), jnp.float32),
            pltpu.VMEM((b_per_w,), jnp.float32),
            pltpu.SemaphoreType.DMA,
            pltpu.SemaphoreType.DMA,
        ],
    )
    def k(user_hbm, recipe_hbm, p_hbm, q_hbm, out_hbm,
          uidx_v, ridx_v, urow_v, rrow_v, pu_v, qr_v, out_v, sem_u, sem_r):
        wid = lax.axis_index("s") * NC + lax.axis_index("c")
        base = wid * b_per_w
        pltpu.sync_copy(user_hbm.at[pl.ds(base, b_per_w)], uidx_v)
        pltpu.sync_copy(recipe_hbm.at[pl.ds(base, b_per_w)], ridx_v)

        def rows(j, carry):
            sl = pl.ds(j * L, L)
            urow_v[sl] = lax.shift_right_logical(uidx_v[sl], 4)
            rrow_v[sl] = lax.shift_right_logical(ridx_v[sl], 4)
            return carry

        lax.fori_loop(0, nblk, rows, 0)
        cp_u = pltpu.async_copy(p_hbm.at[urow_v], pu_v, sem_u)
        cp_r = pltpu.async_copy(q_hbm.at[rrow_v], qr_v, sem_r)
        cp_u.wait()
        cp_r.wait()

        def blk(j, carry):
            sl = pl.ds(j * L, L)
            lanes = j * L + lax.iota(jnp.int32, L)
            ucol = uidx_v[sl] & (L - 1)
            rcol = ridx_v[sl] & (L - 1)
            s = (plsc.load_gather(pu_v, [lanes, ucol])
                 + plsc.load_gather(qr_v, [lanes, rcol]))
            out_v[sl] = 1.0 / (1.0 + jnp.exp(-s))
            return carry

        lax.fori_loop(0, nblk, blk, 0)
        pltpu.sync_copy(out_v, out_hbm.at[pl.ds(base, b_per_w)])

    return k(user, recipe, p2, q2)


def kernel(user, recipe, user_table, recipe_table, W1, b1, W2, b2, W3, b3):
    params = _fold_params(W1, b1, W2, b2, W3, b3)
    p2 = _table_dot(user_table, params, 0, True, 1024)
    q2 = _table_dot(recipe_table, params, 16, False, 1024)
    out = _sc_lookup(user.astype(jnp.int32), recipe.astype(jnp.int32), p2, q2)
    return out.reshape(-1, 1)


# speedup vs baseline: 1.0882x; 1.0882x over previous
import functools

import jax
import jax.numpy as jnp
from jax import lax
from jax.experimental import pallas as pl
from jax.experimental.pallas import tpu as pltpu
from jax.experimental.pallas import tpu_sc as plsc


def _fold_params(W1, b1, W2, b2, W3, b3):
    def body(w1_ref, b1_ref, w2_ref, b2_ref, w3_ref, b3_ref, out_ref):
        w3 = w3_ref[...][:, 0]
        w23 = jnp.sum(w2_ref[...] * w3[None, :], axis=1)
        w = jnp.sum(w1_ref[...] * w23[None, :], axis=1)
        c = (jnp.sum(b1_ref[...] * w23) + jnp.sum(b2_ref[...] * w3)
             + b3_ref[0])
        out_ref[...] = jnp.concatenate(
            [w, jnp.full((16,), c, jnp.float32)])

    return pl.pallas_call(
        body,
        out_shape=jax.ShapeDtypeStruct((48,), jnp.float32),
    )(W1, b1, W2, b2, W3, b3)


def _table_dot(table, params, w_off, add_c, block_groups):
    n = table.shape[0]
    g = n // 16
    rows = block_groups * 16
    grid = (n + rows - 1) // rows

    def body(t_ref, p_ref, out_ref):
        w = p_ref[pl.ds(w_off, 16)]
        s = jnp.dot(t_ref[...], w[:, None],
                    preferred_element_type=jnp.float32)
        if add_c:
            s = s + p_ref[32]
        out_ref[...] = s.reshape(block_groups, 16)

    return pl.pallas_call(
        body,
        grid=(grid,),
        in_specs=[
            pl.BlockSpec((rows, 16), lambda i: (i, 0)),
            pl.BlockSpec((48,), lambda i: (0,)),
        ],
        out_specs=pl.BlockSpec((block_groups, 16), lambda i: (i, 0)),
        out_shape=jax.ShapeDtypeStruct((g, 16), jnp.float32),
    )(table, params)


def _sc_lookup(user, recipe, p2, q2):
    info = plsc.get_sparse_core_info()
    NC, NS, L = info.num_cores, info.num_subcores, info.num_lanes
    NW = NC * NS
    B = user.shape[0]
    b_per_w = B // NW
    nblk = b_per_w // L
    mesh = plsc.VectorSubcoreMesh(core_axis_name="c", subcore_axis_name="s")

    @functools.partial(
        pl.kernel,
        mesh=mesh,
        out_type=jax.ShapeDtypeStruct((B,), jnp.float32),
        compiler_params=pltpu.CompilerParams(
            needs_layout_passes=False, use_tc_tiling_on_sc=False),
        scratch_types=[
            pltpu.VMEM((b_per_w,), jnp.int32),
            pltpu.VMEM((b_per_w,), jnp.int32),
            pltpu.VMEM((b_per_w,), jnp.int32),
            pltpu.VMEM((b_per_w,), jnp.int32),
            pltpu.VMEM((b_per_w, L), jnp.float32),
            pltpu.VMEM((b_per_w, L), jnp.float32),
            pltpu.VMEM((b_per_w,), jnp.float32),
            pltpu.SemaphoreType.DMA,
            pltpu.SemaphoreType.DMA,
        ],
    )
    def k(user_hbm, recipe_hbm, p_hbm, q_hbm, out_hbm,
          uidx_v, ridx_v, urow_v, rrow_v, pu_v, qr_v, out_v, sem_u, sem_r):
        wid = lax.axis_index("s") * NC + lax.axis_index("c")
        base = wid * b_per_w
        pltpu.sync_copy(user_hbm.at[pl.ds(base, b_per_w)], uidx_v)
        pltpu.sync_copy(recipe_hbm.at[pl.ds(base, b_per_w)], ridx_v)

        def rows(j, carry):
            sl = pl.ds(j * L, L)
            urow_v[sl] = lax.shift_right_logical(uidx_v[sl], 4)
            rrow_v[sl] = lax.shift_right_logical(ridx_v[sl], 4)
            return carry

        lax.fori_loop(0, nblk, rows, 0)
        cp_u = pltpu.async_copy(p_hbm.at[urow_v], pu_v, sem_u)
        cp_r = pltpu.async_copy(q_hbm.at[rrow_v], qr_v, sem_r)
        cp_u.wait()
        cp_r.wait()

        def blk(j, carry):
            sl = pl.ds(j * L, L)
            lanes = j * L + lax.iota(jnp.int32, L)
            ucol = uidx_v[sl] & (L - 1)
            rcol = ridx_v[sl] & (L - 1)
            s = (plsc.load_gather(pu_v, [lanes, ucol])
                 + plsc.load_gather(qr_v, [lanes, rcol]))
            out_v[sl] = 1.0 / (1.0 + jnp.exp(-s))
            return carry

        lax.fori_loop(0, nblk, blk, 0)
        pltpu.sync_copy(out_v, out_hbm.at[pl.ds(base, b_per_w)])

    return k(user, recipe, p2, q2)


def kernel(user, recipe, user_table, recipe_table, W1, b1, W2, b2, W3, b3):
    params = _fold_params(W1, b1, W2, b2, W3, b3)
    p2 = _table_dot(user_table, params, 0, True, 1024)
    q2 = _table_dot(recipe_table, params, 16, False, 1024)
    out = _sc_lookup(user.astype(jnp.int32), recipe.astype(jnp.int32), p2, q2)
    return out.reshape(-1, 1)


# speedup vs baseline: 1.7596x; 1.6170x over previous
import functools

import jax
import jax.numpy as jnp
from jax import lax
from jax.experimental import pallas as pl
from jax.experimental.pallas import tpu as pltpu
from jax.experimental.pallas import tpu_sc as plsc


def _fold_params(W1, b1, W2, b2, W3, b3):
    def body(w1_ref, b1_ref, w2_ref, b2_ref, w3_ref, b3_ref, out_ref):
        w3 = w3_ref[...][:, 0]
        w23 = jnp.sum(w2_ref[...] * w3[None, :], axis=1)
        w = jnp.sum(w1_ref[...] * w23[None, :], axis=1)
        c = (jnp.sum(b1_ref[...] * w23) + jnp.sum(b2_ref[...] * w3)
             + b3_ref[0])
        out_ref[...] = jnp.concatenate(
            [w, jnp.full((16,), c, jnp.float32)])

    return pl.pallas_call(
        body,
        out_shape=jax.ShapeDtypeStruct((48,), jnp.float32),
    )(W1, b1, W2, b2, W3, b3)


def _sc_forward(user, recipe, ut, rt, params):
    info = plsc.get_sparse_core_info()
    NC, NS, L = info.num_cores, info.num_subcores, info.num_lanes
    NW = NC * NS
    B = user.shape[0]
    b_per_w = B // NW
    nblk = b_per_w // L
    mesh = plsc.VectorSubcoreMesh(core_axis_name="c", subcore_axis_name="s")

    @functools.partial(
        pl.kernel,
        mesh=mesh,
        out_type=jax.ShapeDtypeStruct((B,), jnp.float32),
        compiler_params=pltpu.CompilerParams(
            needs_layout_passes=False, use_tc_tiling_on_sc=True),
        scratch_types=[
            pltpu.VMEM((b_per_w,), jnp.int32),
            pltpu.VMEM((b_per_w,), jnp.int32),
            pltpu.VMEM((L, L), jnp.float32),
            pltpu.VMEM((L, L), jnp.float32),
            pltpu.VMEM((48,), jnp.float32),
            pltpu.VMEM((b_per_w,), jnp.float32),
            pltpu.SemaphoreType.DMA,
            pltpu.SemaphoreType.DMA,
        ],
    )
    def k(user_hbm, recipe_hbm, ut_hbm, rt_hbm, params_hbm, out_hbm,
          uidx_v, ridx_v, ubuf, rbuf, params_v, out_v, sem_u, sem_r):
        wid = lax.axis_index("s") * NC + lax.axis_index("c")
        base = wid * b_per_w
        pltpu.sync_copy(params_hbm, params_v)
        pltpu.sync_copy(user_hbm.at[pl.ds(base, b_per_w)], uidx_v)
        pltpu.sync_copy(recipe_hbm.at[pl.ds(base, b_per_w)], ridx_v)
        wu_vec = params_v[pl.ds(0, L)]
        wr_vec = params_v[pl.ds(L, L)]
        c_vec = params_v[pl.ds(2 * L, L)]
        wu = [wu_vec[kk] for kk in range(L)]
        wr = [wr_vec[kk] for kk in range(L)]
        c = c_vec[0]

        def blk(j, carry):
            sl = pl.ds(j * L, L)
            uv = uidx_v[sl]
            rv = ridx_v[sl]
            cps = []
            for l in range(L):
                cps.append(pltpu.async_copy(
                    ut_hbm.at[pl.ds(uv[l], 1)], ubuf.at[pl.ds(l, 1)], sem_u))
                cps.append(pltpu.async_copy(
                    rt_hbm.at[pl.ds(rv[l], 1)], rbuf.at[pl.ds(l, 1)], sem_r))
            for cp in cps:
                cp.wait()
            lanes = lax.iota(jnp.int32, L)
            acc = jnp.full((L,), c, jnp.float32)
            for kk in range(L):
                col = jnp.full((L,), kk, jnp.int32)
                acc = acc + plsc.load_gather(ubuf, [lanes, col]) * wu[kk]
                acc = acc + plsc.load_gather(rbuf, [lanes, col]) * wr[kk]
            out_v[sl] = 1.0 / (1.0 + jnp.exp(-acc))
            return carry

        lax.fori_loop(0, nblk, blk, 0)
        pltpu.sync_copy(out_v, out_hbm.at[pl.ds(base, b_per_w)])

    return k(user, recipe, ut, rt, params)


def kernel(user, recipe, user_table, recipe_table, W1, b1, W2, b2, W3, b3):
    params = _fold_params(W1, b1, W2, b2, W3, b3)
    out = _sc_forward(user.astype(jnp.int32), recipe.astype(jnp.int32),
                      user_table, recipe_table, params)
    return out.reshape(-1, 1)


# speedup vs baseline: 1.8659x; 1.0604x over previous
import functools

import jax
import jax.numpy as jnp
from jax import lax
from jax.experimental import pallas as pl
from jax.experimental.pallas import tpu as pltpu
from jax.experimental.pallas import tpu_sc as plsc


def _fold_params(W1, b1, W2, b2, W3, b3):
    def body(w1_ref, b1_ref, w2_ref, b2_ref, w3_ref, b3_ref, out_ref):
        w3 = w3_ref[...][:, 0]
        w23 = jnp.sum(w2_ref[...] * w3[None, :], axis=1)
        w = jnp.sum(w1_ref[...] * w23[None, :], axis=1)
        c = (jnp.sum(b1_ref[...] * w23) + jnp.sum(b2_ref[...] * w3)
             + b3_ref[0])
        out_ref[...] = jnp.concatenate(
            [w, jnp.full((16,), c, jnp.float32)])

    return pl.pallas_call(
        body,
        out_shape=jax.ShapeDtypeStruct((48,), jnp.float32),
    )(W1, b1, W2, b2, W3, b3)


def _sc_forward(user, recipe, ut, rt, params):
    info = plsc.get_sparse_core_info()
    NC, NS, L = info.num_cores, info.num_subcores, info.num_lanes
    NW = NC * NS
    B = user.shape[0]
    b_per_w = B // NW
    nblk = b_per_w // L
    mesh = plsc.VectorSubcoreMesh(core_axis_name="c", subcore_axis_name="s")

    @functools.partial(
        pl.kernel,
        mesh=mesh,
        out_type=jax.ShapeDtypeStruct((B,), jnp.float32),
        compiler_params=pltpu.CompilerParams(
            needs_layout_passes=False, use_tc_tiling_on_sc=True),
        scratch_types=[
            pltpu.VMEM((b_per_w,), jnp.int32),
            pltpu.VMEM((b_per_w,), jnp.int32),
            pltpu.VMEM((2, 128, L), jnp.float32),
            pltpu.VMEM((2, 128, L), jnp.float32),
            pltpu.VMEM((48,), jnp.float32),
            pltpu.VMEM((b_per_w,), jnp.float32),
            pltpu.SemaphoreType.DMA,
            pltpu.SemaphoreType.DMA,
            pltpu.SemaphoreType.DMA,
            pltpu.SemaphoreType.DMA,
        ],
    )
    def k(user_hbm, recipe_hbm, ut_hbm, rt_hbm, params_hbm, out_hbm,
          uidx_v, ridx_v, ubuf, rbuf, params_v, out_v,
          sem_u0, sem_u1, sem_r0, sem_r1):
        sems_u = (sem_u0, sem_u1)
        sems_r = (sem_r0, sem_r1)
        wid = lax.axis_index("s") * NC + lax.axis_index("c")
        base = wid * b_per_w
        pltpu.sync_copy(params_hbm, params_v)
        pltpu.sync_copy(user_hbm.at[pl.ds(base, b_per_w)], uidx_v)
        pltpu.sync_copy(recipe_hbm.at[pl.ds(base, b_per_w)], ridx_v)
        wu_vec = params_v[pl.ds(0, L)]
        wr_vec = params_v[pl.ds(L, L)]
        c_vec = params_v[pl.ds(2 * L, L)]
        wu = [wu_vec[kk] for kk in range(L)]
        wr = [wr_vec[kk] for kk in range(L)]
        c = c_vec[0]

        C = 128
        nch = b_per_w // C
        cblk = C // L

        def make_issue(cidx, slot):
            def issue(j, carry):
                gl = cidx * C + j * L
                uv = uidx_v[pl.ds(gl, L)]
                rv = ridx_v[pl.ds(gl, L)]
                for l in range(L):
                    pltpu.async_copy(
                        ut_hbm.at[pl.ds(uv[l], 1)],
                        ubuf.at[slot, pl.ds(j * L + l, 1)], sems_u[slot])
                    pltpu.async_copy(
                        rt_hbm.at[pl.ds(rv[l], 1)],
                        rbuf.at[slot, pl.ds(j * L + l, 1)], sems_r[slot])
                return carry
            lax.fori_loop(0, cblk, issue, 0)

        slot_full = [jnp.full((L,), s, jnp.int32) for s in (0, 1)]
        lanes0 = lax.iota(jnp.int32, L)

        make_issue(0, 0)
        for cidx in range(nch):
            slot = cidx & 1
            if cidx + 1 < nch:
                make_issue(cidx + 1, (cidx + 1) & 1)
            pltpu.make_async_copy(
                ut_hbm.at[pl.ds(0, C)], ubuf.at[slot], sems_u[slot]).wait()
            pltpu.make_async_copy(
                rt_hbm.at[pl.ds(0, C)], rbuf.at[slot], sems_r[slot]).wait()

            def blk(j, carry, cidx=cidx, slot=slot):
                lanes = j * L + lanes0
                acc = jnp.full((L,), c, jnp.float32)
                for kk in range(L):
                    col = jnp.full((L,), kk, jnp.int32)
                    acc = acc + plsc.load_gather(
                        ubuf, [slot_full[slot], lanes, col]) * wu[kk]
                    acc = acc + plsc.load_gather(
                        rbuf, [slot_full[slot], lanes, col]) * wr[kk]
                out_v[pl.ds(cidx * C + j * L, L)] = 1.0 / (1.0 + jnp.exp(-acc))
                return carry

            lax.fori_loop(0, cblk, blk, 0)
        pltpu.sync_copy(out_v, out_hbm.at[pl.ds(base, b_per_w)])

    return k(user, recipe, ut, rt, params)


def kernel(user, recipe, user_table, recipe_table, W1, b1, W2, b2, W3, b3):
    params = _fold_params(W1, b1, W2, b2, W3, b3)
    out = _sc_forward(user.astype(jnp.int32), recipe.astype(jnp.int32),
                      user_table, recipe_table, params)
    return out.reshape(-1, 1)
